# Initial kernel scaffold; baseline (speedup 1.0000x reference)
import functools

import jax
import jax.numpy as jnp
from jax import lax
from jax.experimental import pallas as pl
from jax.experimental.pallas import tpu as pltpu
from jax.experimental.pallas import tpu_sc as plsc

N = 10000
E = 320000
NCORE = 2
NSUB = 16
NW = NCORE * NSUB
CH = 128
NCHUNK = 80
EPT = NCHUNK * CH
EP = EPT * NW
NP = 10016
EPW = E // NW
KPIPE = 4

_mesh = dict(core_axis_name="c", subcore_axis_name="s")


def _make_sc_conv(width):
    zrows = NP // NSUB
    orows = N // NSUB

    @functools.partial(
        pl.kernel,
        out_type=jax.ShapeDtypeStruct((NCORE, N, width), jnp.float32),
        mesh=plsc.VectorSubcoreMesh(**_mesh),
        scratch_types=[
            pltpu.VMEM((NCHUNK, CH), jnp.int32),
            pltpu.VMEM((NCHUNK, CH), jnp.int32),
        ]
        + [pltpu.VMEM((CH, width), jnp.float32) for _ in range(KPIPE)]
        + [pltpu.VMEM_SHARED((NP, width), jnp.float32)]
        + [pltpu.SemaphoreType.DMA for _ in range(KPIPE)],
    )
    def conv(h_hbm, srcp_hbm, dstp_hbm, z_hbm, out_hbm,
             srcv, dstv, *rest):
        bufs = rest[:KPIPE]
        aggsh = rest[KPIPE]
        sems = rest[KPIPE + 1:]
        cid = lax.axis_index("c")
        sid = lax.axis_index("s")
        wid = sid * NCORE + cid
        pltpu.sync_copy(srcp_hbm.at[wid], srcv)
        pltpu.sync_copy(dstp_hbm.at[wid], dstv)
        pltpu.sync_copy(z_hbm.at[pl.ds(sid * zrows, zrows)],
                        aggsh.at[pl.ds(sid * zrows, zrows)])
        plsc.subcore_barrier()
        for k in range(KPIPE):
            pltpu.async_copy(h_hbm.at[srcv.at[k]], bufs[k], sems[k])

        def body(t, carry):
            base = t * KPIPE
            for k in range(KPIPE):
                pltpu.make_async_copy(h_hbm.at[srcv.at[base + k]],
                                      bufs[k], sems[k]).wait()
                pltpu.sync_copy(bufs[k], aggsh.at[dstv.at[base + k]],
                                add=True)
                nxt = base + k + KPIPE

                @pl.when(nxt < NCHUNK)
                def _():
                    pltpu.async_copy(h_hbm.at[srcv.at[nxt]], bufs[k],
                                     sems[k])
            return carry

        lax.fori_loop(0, NCHUNK // KPIPE, body, 0)
        plsc.subcore_barrier()
        pltpu.sync_copy(aggsh.at[pl.ds(sid * orows, orows)],
                        out_hbm.at[cid, pl.ds(sid * orows, orows)])

    return conv


_sc_conv32 = _make_sc_conv(32)
_sc_conv128 = _make_sc_conv(128)


@functools.partial(
    pl.kernel,
    out_type=(jax.ShapeDtypeStruct((NW, EPW), jnp.float32),
              jax.ShapeDtypeStruct((NW, EPW), jnp.float32)),
    mesh=plsc.VectorSubcoreMesh(**_mesh),
    scratch_types=[
        pltpu.VMEM((EPW,), jnp.int32),
        pltpu.VMEM((EPW,), jnp.int32),
        pltpu.VMEM((N,), jnp.float32),
        pltpu.VMEM((N,), jnp.float32),
        pltpu.VMEM((EPW,), jnp.float32),
        pltpu.VMEM((EPW,), jnp.float32),
    ],
)
def _sc_head(sl_hbm, sr_hbm, src_hbm, dst_hbm, raw_hbm, sig_hbm,
             srcv, dstv, slv, srv, rawv, sigv):
    cid = lax.axis_index("c")
    sid = lax.axis_index("s")
    wid = sid * NCORE + cid
    pltpu.sync_copy(src_hbm.at[wid], srcv)
    pltpu.sync_copy(dst_hbm.at[wid], dstv)
    pltpu.sync_copy(sl_hbm, slv)
    pltpu.sync_copy(sr_hbm, srv)

    def body(i, carry):
        o = i * 16
        s = srcv[pl.ds(o, 16)]
        d = dstv[pl.ds(o, 16)]
        raw = plsc.load_gather(slv, [s]) + plsc.load_gather(srv, [d])
        rawv[pl.ds(o, 16)] = raw
        sigv[pl.ds(o, 16)] = 1.0 / (1.0 + jnp.exp(-raw))
        return carry

    lax.fori_loop(0, EPW // 16, body, 0)
    pltpu.sync_copy(rawv, raw_hbm.at[wid])
    pltpu.sync_copy(sigv, sig_hbm.at[wid])


def _tc_lin0_body(x_ref, w_ref, b_ref, o_ref):
    o_ref[...] = (jnp.dot(x_ref[...], w_ref[...],
                          preferred_element_type=jnp.float32)
                  + b_ref[...][None, :])


def _tc_mid_body(p_ref, h0_ref, wl_ref, wr_ref, b_ref, h1_ref, cnt_ref):
    p = p_ref[0] + p_ref[1]
    cnt = jnp.maximum(p[:, 16:17], 1.0)
    mean = p[:, :16] / cnt
    h1 = (jnp.dot(mean, wl_ref[...], preferred_element_type=jnp.float32)
          + jnp.dot(h0_ref[...][:, :16], wr_ref[...],
                    preferred_element_type=jnp.float32)
          + b_ref[...][None, :])
    h1_ref[...] = jnp.maximum(h1, 0.0)
    cnt_ref[...] = cnt


def _tc_out_body(p_ref, cnt_ref, h1_ref, wl_ref, wr_ref, b_ref,
                 wpl_ref, wpr_ref, bp_ref, sl_ref, sr_ref):
    mean = (p_ref[0] + p_ref[1]) / cnt_ref[...]
    h2 = (jnp.dot(mean, wl_ref[...], preferred_element_type=jnp.float32)
          + jnp.dot(h1_ref[...], wr_ref[...],
                    preferred_element_type=jnp.float32)
          + b_ref[...][None, :])
    sl_ref[...] = (jnp.dot(h2, wpl_ref[...],
                           preferred_element_type=jnp.float32)
                   + bp_ref[...][None, :])
    sr_ref[...] = jnp.dot(h2, wpr_ref[...],
                          preferred_element_type=jnp.float32)


def kernel(x, edge_index, edge_attr, W0, b0, Wl1, Wr1, b1,
           Wl2, Wr2, b2, Wp, bp):
    f32 = jnp.float32
    src = edge_index[0]
    dst = edge_index[1]

    w0aug = jnp.zeros((128, 32), f32).at[:, :16].set(W0.T)
    b0aug = jnp.zeros((32,), f32).at[:16].set(b0).at[16].set(1.0)
    h0aug = pl.pallas_call(
        _tc_lin0_body,
        out_shape=jax.ShapeDtypeStruct((N, 32), f32),
    )(x, w0aug, b0aug)

    pad = EP - E
    srcp = jnp.concatenate([src, jnp.zeros((pad,), jnp.int32)]
                           ).reshape(NW, NCHUNK, CH)
    dstp = jnp.concatenate([dst, jnp.full((pad,), N, jnp.int32)]
                           ).reshape(NW, NCHUNK, CH)

    parts1 = _sc_conv32(h0aug, srcp, dstp, jnp.zeros((NP, 32), f32))

    h1, cntcol = pl.pallas_call(
        _tc_mid_body,
        out_shape=(jax.ShapeDtypeStruct((N, 128), f32),
                   jax.ShapeDtypeStruct((N, 1), f32)),
    )(parts1, h0aug, Wl1.T, Wr1.T, b1)

    parts2 = _sc_conv128(h1, srcp, dstp, jnp.zeros((NP, 128), f32))

    sl, sr = pl.pallas_call(
        _tc_out_body,
        out_shape=(jax.ShapeDtypeStruct((N, 1), f32),
                   jax.ShapeDtypeStruct((N, 1), f32)),
    )(parts2, cntcol, h1, Wl2.T, Wr2.T, b2,
      Wp[0, :128].reshape(128, 1), Wp[0, 128:].reshape(128, 1), bp)

    raw2, sig2 = _sc_head(sl.reshape(N), sr.reshape(N),
                          src.reshape(NW, EPW), dst.reshape(NW, EPW))
    return raw2.reshape(E), sig2.reshape(E)


# speedup vs baseline: 10.9214x; 10.9214x over previous
import functools

import jax
import jax.numpy as jnp
from jax import lax
from jax.experimental import pallas as pl
from jax.experimental.pallas import tpu as pltpu
from jax.experimental.pallas import tpu_sc as plsc

N = 10000
E = 320000
NCORE = 2
NSUB = 16
NW = NCORE * NSUB
CH = 128
NCHUNK = 160
EP = NSUB * NCHUNK * CH
NP = 10112
EPW = E // NW
KPIPE = 4
ZROWS = NP // NSUB
OROWS = 624
OTAIL = N - OROWS * NSUB

_mesh = plsc.VectorSubcoreMesh(core_axis_name="c", subcore_axis_name="s")
_params = pltpu.CompilerParams(use_tc_tiling_on_sc=False)
_params_head = pltpu.CompilerParams(use_tc_tiling_on_sc=False,
                                    needs_layout_passes=False)


def _stage_and_zero(srcp_hbm, dstp_hbm, z_hbm, srcv, dstv, aggsh, sid):
    pltpu.sync_copy(srcp_hbm.at[sid], srcv)
    pltpu.sync_copy(dstp_hbm.at[sid], dstv)
    pltpu.sync_copy(z_hbm.at[pl.ds(sid * ZROWS, ZROWS)],
                    aggsh.at[pl.ds(sid * ZROWS, ZROWS)])


def _copy_out(aggsh, out_hbm, cid, sid):
    pltpu.sync_copy(aggsh.at[pl.ds(sid * OROWS, OROWS)],
                    out_hbm.at[cid, pl.ds(sid * OROWS, OROWS)])

    @pl.when(sid == 0)
    def _():
        pltpu.sync_copy(aggsh.at[pl.ds(OROWS * NSUB, OTAIL)],
                        out_hbm.at[cid, pl.ds(OROWS * NSUB, OTAIL)])


def _gather_scatter_loop(src_hbm, srcv, dstv, bufs, sems, aggsh):
    for k in range(KPIPE):
        pltpu.async_copy(src_hbm.at[srcv.at[k]], bufs[k], sems[k])

    def body(t, carry):
        base = t * KPIPE
        for k in range(KPIPE):
            pltpu.make_async_copy(src_hbm.at[srcv.at[base + k]],
                                  bufs[k], sems[k]).wait()
            pltpu.sync_copy(bufs[k], aggsh.at[dstv.at[base + k]], add=True)
            nxt = base + k + KPIPE

            @pl.when(nxt < NCHUNK)
            def _():
                pltpu.async_copy(src_hbm.at[srcv.at[nxt]], bufs[k], sems[k])
        return carry

    lax.fori_loop(0, NCHUNK // KPIPE, body, 0)


@functools.partial(
    pl.kernel,
    out_type=jax.ShapeDtypeStruct((NCORE, N, 16), jnp.float32),
    mesh=_mesh,
    scratch_types=[
        pltpu.VMEM((NCHUNK, CH), jnp.int32),
        pltpu.VMEM((NCHUNK, CH), jnp.int32),
    ]
    + [pltpu.VMEM((CH, 16), jnp.float32) for _ in range(KPIPE)]
    + [pltpu.VMEM_SHARED((NP, 16), jnp.float32)]
    + [pltpu.SemaphoreType.DMA for _ in range(KPIPE)],
    compiler_params=_params,
)
def _sc_conv1(h0_hbm, srcp_hbm, dstp_hbm, z_hbm, out_hbm,
              srcv, dstv, *rest):
    bufs = rest[:KPIPE]
    aggsh = rest[KPIPE]
    sems = rest[KPIPE + 1:]
    cid = lax.axis_index("c")
    sid = lax.axis_index("s")
    _stage_and_zero(srcp_hbm, dstp_hbm, z_hbm, srcv, dstv, aggsh, sid)
    plsc.subcore_barrier()

    @pl.when(cid == 0)
    def _():
        _gather_scatter_loop(h0_hbm, srcv, dstv, bufs, sems, aggsh)

    @pl.when(cid == 1)
    def _():
        ones_col = jnp.where(lax.iota(jnp.int32, 16) == 0, 1.0, 0.0)

        def fill(r, carry):
            bufs[0][r] = ones_col
            return carry

        lax.fori_loop(0, CH, fill, 0)

        def body(t, carry):
            pltpu.sync_copy(bufs[0], aggsh.at[dstv.at[t]], add=True)
            return carry

        lax.fori_loop(0, NCHUNK, body, 0)

    plsc.subcore_barrier()
    _copy_out(aggsh, out_hbm, cid, sid)


@functools.partial(
    pl.kernel,
    out_type=jax.ShapeDtypeStruct((NCORE, N, 64), jnp.float32),
    mesh=_mesh,
    scratch_types=[
        pltpu.VMEM((NCHUNK, CH), jnp.int32),
        pltpu.VMEM((NCHUNK, CH), jnp.int32),
    ]
    + [pltpu.VMEM((CH, 64), jnp.float32) for _ in range(KPIPE)]
    + [pltpu.VMEM_SHARED((NP, 64), jnp.float32)]
    + [pltpu.SemaphoreType.DMA for _ in range(KPIPE)],
    compiler_params=_params,
)
def _sc_conv2(h1p_hbm, srcp_hbm, dstp_hbm, z_hbm, out_hbm,
              srcv, dstv, *rest):
    bufs = rest[:KPIPE]
    aggsh = rest[KPIPE]
    sems = rest[KPIPE + 1:]
    cid = lax.axis_index("c")
    sid = lax.axis_index("s")
    _stage_and_zero(srcp_hbm, dstp_hbm, z_hbm, srcv, dstv, aggsh, sid)
    plsc.subcore_barrier()
    _gather_scatter_loop(h1p_hbm.at[cid], srcv, dstv, bufs, sems, aggsh)
    plsc.subcore_barrier()
    _copy_out(aggsh, out_hbm, cid, sid)


@functools.partial(
    pl.kernel,
    out_type=(jax.ShapeDtypeStruct((NW, EPW), jnp.float32),
              jax.ShapeDtypeStruct((NW, EPW), jnp.float32)),
    mesh=_mesh,
    scratch_types=[
        pltpu.VMEM((EPW,), jnp.int32),
        pltpu.VMEM((EPW,), jnp.int32),
        pltpu.VMEM((N,), jnp.float32),
        pltpu.VMEM((N,), jnp.float32),
        pltpu.VMEM((EPW,), jnp.float32),
        pltpu.VMEM((EPW,), jnp.float32),
    ],
    compiler_params=_params_head,
)
def _sc_head(sl_hbm, sr_hbm, src_hbm, dst_hbm, raw_hbm, sig_hbm,
             srcv, dstv, slv, srv, rawv, sigv):
    cid = lax.axis_index("c")
    sid = lax.axis_index("s")
    wid = sid * NCORE + cid
    pltpu.sync_copy(src_hbm.at[wid], srcv)
    pltpu.sync_copy(dst_hbm.at[wid], dstv)
    pltpu.sync_copy(sl_hbm, slv)
    pltpu.sync_copy(sr_hbm, srv)

    def body(i, carry):
        o = i * 16
        s = srcv[pl.ds(o, 16)]
        d = dstv[pl.ds(o, 16)]
        raw = plsc.load_gather(slv, [s]) + plsc.load_gather(srv, [d])
        rawv[pl.ds(o, 16)] = raw
        sigv[pl.ds(o, 16)] = 1.0 / (1.0 + jnp.exp(-raw))
        return carry

    lax.fori_loop(0, EPW // 16, body, 0)
    pltpu.sync_copy(rawv, raw_hbm.at[wid])
    pltpu.sync_copy(sigv, sig_hbm.at[wid])


def _tc_lin0_body(x_ref, w_ref, b_ref, o_ref):
    o_ref[...] = (jnp.dot(x_ref[...], w_ref[...],
                          preferred_element_type=jnp.float32)
                  + b_ref[...][None, :])


def _tc_mid_body(p_ref, h0_ref, wl_ref, wr_ref, b_ref, h1p_ref, cnt_ref):
    cnt = jnp.maximum(p_ref[1][:, :1], 1.0)
    mean = p_ref[0] / cnt
    h1 = (jnp.dot(mean, wl_ref[...], preferred_element_type=jnp.float32)
          + jnp.dot(h0_ref[...], wr_ref[...],
                    preferred_element_type=jnp.float32)
          + b_ref[...][None, :])
    h1 = jnp.maximum(h1, 0.0)
    h1p_ref[0] = h1[:, :64]
    h1p_ref[1] = h1[:, 64:]
    cnt_ref[...] = cnt


def _tc_out_body(p_ref, cnt_ref, h1p_ref, wl_ref, wr_ref, b_ref,
                 wpl_ref, wpr_ref, bp_ref, sl_ref, sr_ref):
    cnt = cnt_ref[...]
    h2 = (jnp.dot(p_ref[0] / cnt, wl_ref[...][:64],
                  preferred_element_type=jnp.float32)
          + jnp.dot(p_ref[1] / cnt, wl_ref[...][64:],
                    preferred_element_type=jnp.float32)
          + jnp.dot(h1p_ref[0], wr_ref[...][:64],
                    preferred_element_type=jnp.float32)
          + jnp.dot(h1p_ref[1], wr_ref[...][64:],
                    preferred_element_type=jnp.float32)
          + b_ref[...][None, :])
    sl_ref[...] = (jnp.dot(h2, wpl_ref[...],
                           preferred_element_type=jnp.float32)
                   + bp_ref[...][None, :])
    sr_ref[...] = jnp.dot(h2, wpr_ref[...],
                          preferred_element_type=jnp.float32)


def kernel(x, edge_index, edge_attr, W0, b0, Wl1, Wr1, b1,
           Wl2, Wr2, b2, Wp, bp):
    f32 = jnp.float32
    src = edge_index[0]
    dst = edge_index[1]

    h0 = pl.pallas_call(
        _tc_lin0_body,
        out_shape=jax.ShapeDtypeStruct((N, 16), f32),
    )(x, W0.T, b0)

    pad = EP - E
    srcp = jnp.concatenate([src, jnp.zeros((pad,), jnp.int32)]
                           ).reshape(NSUB, NCHUNK, CH)
    dstp = jnp.concatenate([dst, jnp.full((pad,), N, jnp.int32)]
                           ).reshape(NSUB, NCHUNK, CH)

    parts1 = _sc_conv1(h0, srcp, dstp, jnp.zeros((NP, 16), f32))

    h1p, cntcol = pl.pallas_call(
        _tc_mid_body,
        out_shape=(jax.ShapeDtypeStruct((NCORE, N, 64), f32),
                   jax.ShapeDtypeStruct((N, 1), f32)),
    )(parts1, h0, Wl1.T, Wr1.T, b1)

    parts2 = _sc_conv2(h1p, srcp, dstp, jnp.zeros((NP, 64), f32))

    sl, sr = pl.pallas_call(
        _tc_out_body,
        out_shape=(jax.ShapeDtypeStruct((N, 1), f32),
                   jax.ShapeDtypeStruct((N, 1), f32)),
    )(parts2, cntcol, h1p, Wl2.T, Wr2.T, b2,
      Wp[0, :128].reshape(128, 1), Wp[0, 128:].reshape(128, 1), bp)

    raw2, sig2 = _sc_head(sl.reshape(N), sr.reshape(N),
                          src.reshape(NW, EPW), dst.reshape(NW, EPW))
    return raw2.reshape(E), sig2.reshape(E)
